# baseline (speedup 1.0000x reference)
import jax
import jax.numpy as jnp
from jax import lax
from jax.experimental import pallas as pl
from jax.experimental.pallas import tpu as pltpu

_K = 3
_ITERS = 5
_ROWS = 2048
_COLS = 128


def _kmeans_body(dep_ref, i_ref, v_ref, o_ref):
    f32 = jnp.float32
    x = v_ref[0]
    y = v_ref[1]
    z = v_ref[2]

    nn = f32(_ROWS * _COLS)
    sx_t = jnp.sum(x)
    sy_t = jnp.sum(y)
    sz_t = jnp.sum(z)

    flat = (lax.broadcasted_iota(jnp.int32, (_ROWS, _COLS), 0) * _COLS
            + lax.broadcasted_iota(jnp.int32, (_ROWS, _COLS), 1))
    cinit = []
    for i in range(_K):
        m = flat == i_ref[i]
        zf = f32(0.0)
        cinit += [jnp.sum(jnp.where(m, x, zf)),
                  jnp.sum(jnp.where(m, y, zf)),
                  jnp.sum(jnp.where(m, z, zf))]

    def masks_from(c):
        c0x, c0y, c0z, c1x, c1y, c1z, c2x, c2y, c2z = c
        q0 = c0x * c0x + c0y * c0y + c0z * c0z
        q1 = c1x * c1x + c1y * c1y + c1z * c1z
        q2 = c2x * c2x + c2y * c2y + c2z * c2z
        g0 = x * (-2.0 * c0x) + y * (-2.0 * c0y) + z * (-2.0 * c0z) + q0
        g1 = x * (-2.0 * c1x) + y * (-2.0 * c1y) + z * (-2.0 * c1z) + q1
        g2 = x * (-2.0 * c2x) + y * (-2.0 * c2y) + z * (-2.0 * c2z) + q2
        lt1 = g1 < g0
        not2 = jnp.logical_not(g2 < jnp.minimum(g0, g1))
        sel0 = jnp.logical_and(jnp.logical_not(lt1), not2)
        sel1 = jnp.logical_and(lt1, not2)
        return sel0, sel1

    zero = f32(0.0)
    c = tuple(cinit)
    for _ in range(_ITERS - 1):
        sel0, sel1 = masks_from(c)
        n0 = jnp.sum(jnp.where(sel0, 1.0, zero))
        n1 = jnp.sum(jnp.where(sel1, 1.0, zero))
        n2 = nn - n0 - n1
        sx0 = jnp.sum(jnp.where(sel0, x, zero))
        sy0 = jnp.sum(jnp.where(sel0, y, zero))
        sz0 = jnp.sum(jnp.where(sel0, z, zero))
        sx1 = jnp.sum(jnp.where(sel1, x, zero))
        sy1 = jnp.sum(jnp.where(sel1, y, zero))
        sz1 = jnp.sum(jnp.where(sel1, z, zero))
        c = (sx0 / n0, sy0 / n0, sz0 / n0,
             sx1 / n1, sy1 / n1, sz1 / n1,
             (sx_t - sx0 - sx1) / n2,
             (sy_t - sy0 - sy1) / n2,
             (sz_t - sz0 - sz1) / n2)

    sel0, _ = masks_from(c)
    base = dep_ref[0]
    plane = jnp.where(sel0, zero, base)
    o_ref[0] = plane
    o_ref[1] = plane
    o_ref[2] = plane


def kernel(data, img_shape):
    data = data.reshape((-1, 3))
    n = data.shape[0]
    init_idx = jax.random.randint(jax.random.key(42), (3,), 0, n).astype(jnp.int32)
    dep = ((jnp.asarray(img_shape[0]) + jnp.asarray(img_shape[1])
            + jnp.asarray(img_shape[2])) * 0).astype(data.dtype).reshape(1)
    v = data.T.reshape(3, _ROWS, _COLS)

    out = pl.pallas_call(
        _kmeans_body,
        in_specs=[
            pl.BlockSpec(memory_space=pltpu.SMEM),
            pl.BlockSpec(memory_space=pltpu.SMEM),
            pl.BlockSpec(memory_space=pltpu.VMEM),
        ],
        out_specs=pl.BlockSpec(memory_space=pltpu.VMEM),
        out_shape=jax.ShapeDtypeStruct((3, _ROWS, _COLS), jnp.float32),
    )(dep, init_idx, v)

    return out.reshape(3, n).T.reshape(n, 1, 3)


# speedup vs baseline: 1.0605x; 1.0605x over previous
import jax
import jax.numpy as jnp
from jax import lax
from jax.experimental import pallas as pl
from jax.experimental.pallas import tpu as pltpu

_K = 3
_ITERS = 5
_ROWS = 2048
_COLS = 128


def _kmeans_body(dep_ref, i_ref, v_ref, o_ref):
    f32 = jnp.float32
    x = v_ref[0]
    y = v_ref[1]
    z = v_ref[2]

    nn = f32(_ROWS * _COLS)
    sx_t = jnp.sum(x)
    sy_t = jnp.sum(y)
    sz_t = jnp.sum(z)

    flat = (lax.broadcasted_iota(jnp.int32, (_ROWS, _COLS), 0) * _COLS
            + lax.broadcasted_iota(jnp.int32, (_ROWS, _COLS), 1))
    cinit = []
    for i in range(_K):
        m = flat == i_ref[i]
        zf = f32(0.0)
        cinit += [jnp.sum(jnp.where(m, x, zf)),
                  jnp.sum(jnp.where(m, y, zf)),
                  jnp.sum(jnp.where(m, z, zf))]

    def masks_from(c):
        c0x, c0y, c0z, c1x, c1y, c1z, c2x, c2y, c2z = c
        q0 = c0x * c0x + c0y * c0y + c0z * c0z
        q1 = c1x * c1x + c1y * c1y + c1z * c1z
        q2 = c2x * c2x + c2y * c2y + c2z * c2z
        h1 = (x * (2.0 * (c0x - c1x)) + y * (2.0 * (c0y - c1y))
              + z * (2.0 * (c0z - c1z)) + (q1 - q0))
        h2 = (x * (2.0 * (c0x - c2x)) + y * (2.0 * (c0y - c2y))
              + z * (2.0 * (c0z - c2z)) + (q2 - q0))
        lt1 = h1 < 0.0
        not2 = jnp.logical_not(h2 < jnp.minimum(h1, 0.0))
        sel0 = jnp.logical_and(jnp.logical_not(lt1), not2)
        sel1 = jnp.logical_and(lt1, not2)
        return sel0, sel1

    zero = f32(0.0)
    c = tuple(cinit)
    for _ in range(_ITERS - 1):
        sel0, sel1 = masks_from(c)
        n0 = jnp.sum(jnp.where(sel0, 1.0, zero))
        n1 = jnp.sum(jnp.where(sel1, 1.0, zero))
        n2 = nn - n0 - n1
        sx0 = jnp.sum(jnp.where(sel0, x, zero))
        sy0 = jnp.sum(jnp.where(sel0, y, zero))
        sz0 = jnp.sum(jnp.where(sel0, z, zero))
        sx1 = jnp.sum(jnp.where(sel1, x, zero))
        sy1 = jnp.sum(jnp.where(sel1, y, zero))
        sz1 = jnp.sum(jnp.where(sel1, z, zero))
        c = (sx0 / n0, sy0 / n0, sz0 / n0,
             sx1 / n1, sy1 / n1, sz1 / n1,
             (sx_t - sx0 - sx1) / n2,
             (sy_t - sy0 - sy1) / n2,
             (sz_t - sz0 - sz1) / n2)

    sel0, _ = masks_from(c)
    base = dep_ref[0]
    plane = jnp.where(sel0, zero, base)
    o_ref[0] = plane
    o_ref[1] = plane
    o_ref[2] = plane


def kernel(data, img_shape):
    data = data.reshape((-1, 3))
    n = data.shape[0]
    init_idx = jax.random.randint(jax.random.key(42), (3,), 0, n).astype(jnp.int32)
    dep = ((jnp.asarray(img_shape[0]) + jnp.asarray(img_shape[1])
            + jnp.asarray(img_shape[2])) * 0).astype(data.dtype).reshape(1)
    v = data.T.reshape(3, _ROWS, _COLS)

    out = pl.pallas_call(
        _kmeans_body,
        in_specs=[
            pl.BlockSpec(memory_space=pltpu.SMEM),
            pl.BlockSpec(memory_space=pltpu.SMEM),
            pl.BlockSpec(memory_space=pltpu.VMEM),
        ],
        out_specs=pl.BlockSpec(memory_space=pltpu.VMEM),
        out_shape=jax.ShapeDtypeStruct((3, _ROWS, _COLS), jnp.float32),
    )(dep, init_idx, v)

    return out.reshape(3, n).T.reshape(n, 1, 3)


# speedup vs baseline: 1.2414x; 1.1706x over previous
import jax
import jax.numpy as jnp
from jax import lax
from jax.experimental import pallas as pl
from jax.experimental.pallas import tpu as pltpu

_K = 3
_ITERS = 5
_ROWS = 2048
_COLS = 128


def _kmeans_body(dep_ref, i_ref, v_ref, o_ref):
    f32 = jnp.float32
    x = v_ref[0]
    y = v_ref[1]
    z = v_ref[2]

    nn = f32(_ROWS * _COLS)
    sx_t = jnp.sum(x)
    sy_t = jnp.sum(y)
    sz_t = jnp.sum(z)

    flat = (lax.broadcasted_iota(jnp.int32, (_ROWS, _COLS), 0) * _COLS
            + lax.broadcasted_iota(jnp.int32, (_ROWS, _COLS), 1))
    cinit = []
    for i in range(_K):
        m = flat == i_ref[i]
        zf = f32(0.0)
        cinit += [jnp.sum(jnp.where(m, x, zf)),
                  jnp.sum(jnp.where(m, y, zf)),
                  jnp.sum(jnp.where(m, z, zf))]

    def masks_from(c):
        c0x, c0y, c0z, c1x, c1y, c1z, c2x, c2y, c2z = c
        q0 = c0x * c0x + c0y * c0y + c0z * c0z
        q1 = c1x * c1x + c1y * c1y + c1z * c1z
        q2 = c2x * c2x + c2y * c2y + c2z * c2z
        h1 = (x * (2.0 * (c0x - c1x)) + y * (2.0 * (c0y - c1y))
              + z * (2.0 * (c0z - c1z)) + (q1 - q0))
        h2 = (x * (2.0 * (c0x - c2x)) + y * (2.0 * (c0y - c2y))
              + z * (2.0 * (c0z - c2z)) + (q2 - q0))
        lt1 = h1 < 0.0
        not2 = jnp.logical_not(h2 < jnp.minimum(h1, 0.0))
        sel0 = jnp.logical_and(jnp.logical_not(lt1), not2)
        sel1 = jnp.logical_and(lt1, not2)
        return sel0, sel1

    zero = f32(0.0)
    c = tuple(cinit)
    for _ in range(_ITERS - 1):
        sel0, sel1 = masks_from(c)
        n0 = jnp.sum(jnp.where(sel0, 1.0, zero))
        n1 = jnp.sum(jnp.where(sel1, 1.0, zero))
        n2 = nn - n0 - n1
        sx0 = jnp.sum(jnp.where(sel0, x, zero))
        sy0 = jnp.sum(jnp.where(sel0, y, zero))
        sz0 = jnp.sum(jnp.where(sel0, z, zero))
        sx1 = jnp.sum(jnp.where(sel1, x, zero))
        sy1 = jnp.sum(jnp.where(sel1, y, zero))
        sz1 = jnp.sum(jnp.where(sel1, z, zero))
        c = (sx0 / n0, sy0 / n0, sz0 / n0,
             sx1 / n1, sy1 / n1, sz1 / n1,
             (sx_t - sx0 - sx1) / n2,
             (sy_t - sy0 - sy1) / n2,
             (sz_t - sz0 - sz1) / n2)

    sel0, _ = masks_from(c)
    base = dep_ref[0]
    o_ref[...] = jnp.where(sel0, zero, base)


def kernel(data, img_shape):
    data = data.reshape((-1, 3))
    n = data.shape[0]
    init_idx = jax.random.randint(jax.random.key(42), (3,), 0, n).astype(jnp.int32)
    dep = ((jnp.asarray(img_shape[0]) + jnp.asarray(img_shape[1])
            + jnp.asarray(img_shape[2])) * 0).astype(data.dtype).reshape(1)
    v = data.T.reshape(3, _ROWS, _COLS)

    out = pl.pallas_call(
        _kmeans_body,
        in_specs=[
            pl.BlockSpec(memory_space=pltpu.SMEM),
            pl.BlockSpec(memory_space=pltpu.SMEM),
            pl.BlockSpec(memory_space=pltpu.VMEM),
        ],
        out_specs=pl.BlockSpec(memory_space=pltpu.VMEM),
        out_shape=jax.ShapeDtypeStruct((_ROWS, _COLS), jnp.float32),
    )(dep, init_idx, v)

    return jnp.broadcast_to(out.reshape(n, 1, 1), (n, 1, 3))
